# Initial kernel scaffold; baseline (speedup 1.0000x reference)
import jax
import jax.numpy as jnp
from jax.experimental import pallas as pl


def kernel(h_prob, edge_index, edge_type, r_index):
    raise NotImplementedError("write your pallas kernel here")


# speedup vs baseline: 38.8963x; 38.8963x over previous
import functools

import jax
import jax.numpy as jnp
from jax import lax
from jax.experimental import pallas as pl
from jax.experimental.pallas import tpu as pltpu
from jax.experimental.pallas import tpu_sc as plsc

BATCH = 8
NNODES = 100000
NEDGES = 6400000

NCORES = 2
NSUB = 16
BPC = BATCH // NCORES
EPT = NEDGES // NSUB
CH = 2000
NCH = EPT // CH
VPC = CH // 16
CAP = 8192
GC = 128
NSEG = 4
SEG = NNODES // NSEG
ACCW = 25008
SL = 1568
LAST_OFF = SEG - SL


def _sc_traversal(h_flat, src, dst, edge_type, r16):
    mesh = plsc.VectorSubcoreMesh(core_axis_name="c", subcore_axis_name="s")

    @functools.partial(
        pl.kernel,
        mesh=mesh,
        out_type=jax.ShapeDtypeStruct((BATCH * NNODES,), jnp.float32),
        compiler_params=pltpu.CompilerParams(needs_layout_passes=False),
        scratch_types=[
            pltpu.VMEM((CH,), jnp.int32),
            pltpu.VMEM((CAP + 16,), jnp.int32),
            pltpu.VMEM((CAP + 16,), jnp.int32),
            pltpu.VMEM((CAP + 16,), jnp.int32),
            pltpu.VMEM((CAP + 16,), jnp.int32),
            pltpu.VMEM((CAP,), jnp.int32),
            pltpu.VMEM((CAP,), jnp.int32),
            pltpu.VMEM((CAP,), jnp.float32),
            pltpu.VMEM((ACCW,), jnp.float32),
            pltpu.VMEM((SL,), jnp.float32),
            pltpu.VMEM((SL,), jnp.float32),
            pltpu.VMEM((16,), jnp.int32),
            pltpu.VMEM_SHARED((NSUB * ACCW,), jnp.float32),
            pltpu.SemaphoreType.DMA,
        ],
    )
    def body(h_hbm, src_hbm, dst_hbm, et_hbm, r_hbm, out_hbm,
             et_buf, l0, l1, l2, l3, srcb, dstb, hb, acc, red, tmp, rv,
             shared, sem):
        c = lax.axis_index("c")
        s = lax.axis_index("s")
        lists = [l0, l1, l2, l3]
        iota16 = lax.iota(jnp.int32, 16)
        zeros16 = jnp.zeros((16,), jnp.float32)

        pltpu.sync_copy(r_hbm, rv)

        def zlist_body(j, _):
            for li in lists:
                li[pl.ds(j * 16, 16)] = jnp.zeros((16,), jnp.int32)
            return 0
        lax.fori_loop(0, (CAP + 16) // 16, zlist_body, 0)

        rb = [plsc.load_gather(rv, [jnp.zeros((16,), jnp.int32) + (BPC * c + i)])
              for i in range(BPC)]

        ebase = s * EPT

        def chunk_body(ci, offs):
            base = pl.multiple_of(ebase + ci * CH, 8)
            pltpu.sync_copy(et_hbm.at[pl.ds(base, CH)], et_buf)

            def vec_body(j, offs):
                t = et_buf[pl.ds(j * 16, 16)]
                gid = base + j * 16 + iota16
                new = []
                for i in range(BPC):
                    m = t == rb[i]
                    cnt = jnp.sum(m.astype(jnp.int32))
                    plsc.store_compressed(lists[i].at[pl.ds(offs[i], 16)], gid, mask=m)
                    new.append(jnp.minimum(offs[i] + cnt, CAP))
                return tuple(new)

            return lax.fori_loop(0, VPC, vec_body, offs)

        z = jnp.int32(0)
        offs = lax.fori_loop(0, NCH, chunk_body, (z, z, z, z))

        for i in range(BPC):
            b = BPC * c + i
            nb = offs[i]
            li = lists[i]
            nch = (nb + GC - 1) // GC

            def fire_sd(k, _):
                idx = li.at[pl.ds(k * GC, GC)]
                pltpu.make_async_copy(src_hbm.at[idx], srcb.at[pl.ds(k * GC, GC)], sem).start()
                pltpu.make_async_copy(dst_hbm.at[idx], dstb.at[pl.ds(k * GC, GC)], sem).start()
                return 0

            def drain_sd(k, _):
                idx = li.at[pl.ds(k * GC, GC)]
                pltpu.make_async_copy(src_hbm.at[idx], srcb.at[pl.ds(k * GC, GC)], sem).wait()
                pltpu.make_async_copy(dst_hbm.at[idx], dstb.at[pl.ds(k * GC, GC)], sem).wait()
                return 0

            lax.fori_loop(0, nch, fire_sd, 0)
            lax.fori_loop(0, nch, drain_sd, 0)

            boff = b * NNODES

            def abs_body(j, _):
                srcb[pl.ds(j * 16, 16)] = srcb[pl.ds(j * 16, 16)] + boff
                return 0

            lax.fori_loop(0, (nb + 15) // 16, abs_body, 0)

            def fire_h(k, _):
                idx = srcb.at[pl.ds(k * GC, GC)]
                pltpu.make_async_copy(h_hbm.at[idx], hb.at[pl.ds(k * GC, GC)], sem).start()
                return 0

            def drain_h(k, _):
                idx = srcb.at[pl.ds(k * GC, GC)]
                pltpu.make_async_copy(h_hbm.at[idx], hb.at[pl.ds(k * GC, GC)], sem).wait()
                return 0

            lax.fori_loop(0, nch, fire_h, 0)
            lax.fori_loop(0, nch, drain_h, 0)

            for seg in range(NSEG):
                lo = seg * SEG

                def zacc_body(j, _):
                    acc[pl.ds(j * 16, 16)] = zeros16
                    return 0

                lax.fori_loop(0, ACCW // 16, zacc_body, 0)

                def scat_body(j, _):
                    d = dstb[pl.ds(j * 16, 16)]
                    v = hb[pl.ds(j * 16, 16)]
                    valid = (j * 16 + iota16 < nb) & (d >= lo) & (d < lo + SEG)
                    loc = jnp.where(valid, d - lo, 0)
                    veff = jnp.where(valid, v, -1.0)

                    def wbody(_):
                        cur = plsc.load_gather(acc, [loc])
                        upd = veff > cur
                        plsc.store_scatter(acc, [loc], veff, mask=upd)
                        cur2 = plsc.load_gather(acc, [loc])
                        return jnp.any(veff > cur2)

                    lax.while_loop(lambda p: p, wbody, jnp.bool_(True))
                    return 0

                lax.fori_loop(0, (nb + 15) // 16, scat_body, 0)

                pltpu.sync_copy(acc, shared.at[pl.ds(pl.multiple_of(s * ACCW, 8), ACCW)])
                plsc.subcore_barrier()

                roff = pl.multiple_of(jnp.where(s < NSUB - 1, s * SL, LAST_OFF), 8)
                pltpu.sync_copy(shared.at[pl.ds(roff, SL)], red)
                for t in range(1, NSUB):
                    pltpu.sync_copy(shared.at[pl.ds(pl.multiple_of(t * ACCW + roff, 8), SL)], tmp)

                    def rmax_body(j, _):
                        red[pl.ds(j * 16, 16)] = jnp.maximum(
                            red[pl.ds(j * 16, 16)], tmp[pl.ds(j * 16, 16)])
                        return 0

                    lax.fori_loop(0, SL // 16, rmax_body, 0)

                out_off = pl.multiple_of(b * NNODES + lo + roff, 8)
                pltpu.sync_copy(red, out_hbm.at[pl.ds(out_off, SL)])
                plsc.subcore_barrier()

    return body(h_flat, src, dst, edge_type, r16)


def kernel(h_prob, edge_index, edge_type, r_index):
    h_flat = h_prob.reshape(-1)
    src = edge_index[0]
    dst = edge_index[1]
    r16 = jnp.concatenate([r_index, jnp.zeros((16 - BATCH,), jnp.int32)])
    out = _sc_traversal(h_flat, src, dst, edge_type, r16)
    return out.reshape(BATCH, NNODES)


# speedup vs baseline: 45.1256x; 1.1602x over previous
import functools

import jax
import jax.numpy as jnp
from jax import lax
from jax.experimental import pallas as pl
from jax.experimental.pallas import tpu as pltpu
from jax.experimental.pallas import tpu_sc as plsc

BATCH = 8
NNODES = 100000
NEDGES = 6400000

NCORES = 2
NSUB = 16
BPC = BATCH // NCORES
EPT = NEDGES // NSUB
CH = 2000
NCH = EPT // CH
VPC = CH // 16
CAP = 8192
GC = 128
NSEG = 4
SEG = NNODES // NSEG
ACCW = 25008
SL = 1568
LAST_OFF = SEG - SL


def _sc_traversal(h_flat, src, dst, edge_type, r16):
    mesh = plsc.VectorSubcoreMesh(core_axis_name="c", subcore_axis_name="s")

    @functools.partial(
        pl.kernel,
        mesh=mesh,
        out_type=jax.ShapeDtypeStruct((BATCH * NNODES,), jnp.float32),
        compiler_params=pltpu.CompilerParams(needs_layout_passes=False),
        scratch_types=[
            pltpu.VMEM((CH,), jnp.int32),
            pltpu.VMEM((CH,), jnp.int32),
            pltpu.VMEM((CAP + 16,), jnp.int32),
            pltpu.VMEM((CAP + 16,), jnp.int32),
            pltpu.VMEM((CAP + 16,), jnp.int32),
            pltpu.VMEM((CAP + 16,), jnp.int32),
            pltpu.VMEM((CAP,), jnp.int32),
            pltpu.VMEM((CAP,), jnp.int32),
            pltpu.VMEM((CAP,), jnp.float32),
            pltpu.VMEM((ACCW,), jnp.float32),
            pltpu.VMEM((SL,), jnp.float32),
            pltpu.VMEM((SL,), jnp.float32),
            pltpu.VMEM((16,), jnp.int32),
            pltpu.VMEM_SHARED((NSUB * ACCW,), jnp.float32),
            pltpu.SemaphoreType.DMA,
            pltpu.SemaphoreType.DMA,
        ],
    )
    def body(h_hbm, src_hbm, dst_hbm, et_hbm, r_hbm, out_hbm,
             et_buf, et_buf2, l0, l1, l2, l3, srcb, dstb, hb, acc, red, tmp, rv,
             shared, sem, sem2):
        c = lax.axis_index("c")
        s = lax.axis_index("s")
        lists = [l0, l1, l2, l3]
        iota16 = lax.iota(jnp.int32, 16)
        zeros16 = jnp.zeros((16,), jnp.float32)

        pltpu.sync_copy(r_hbm, rv)

        def zlist_body(j, _):
            for li in lists:
                li[pl.ds(j * 16, 16)] = jnp.zeros((16,), jnp.int32)
            return 0
        lax.fori_loop(0, (CAP + 16) // 16, zlist_body, 0)

        rb = [plsc.load_gather(rv, [jnp.zeros((16,), jnp.int32) + (BPC * c + i)])
              for i in range(BPC)]

        ebase = s * EPT

        def et_start(buf, ci, sem_):
            base = pl.multiple_of(ebase + ci * CH, 8)
            pltpu.make_async_copy(et_hbm.at[pl.ds(base, CH)], buf, sem_).start()

        def et_wait(buf, sem_):
            pltpu.make_async_copy(et_hbm.at[pl.ds(ebase, CH)], buf, sem_).wait()

        def scan_chunk(buf, ci, offs):
            base = ebase + ci * CH

            def vec_body(j, offs):
                t = buf[pl.ds(j * 16, 16)]
                gid = base + j * 16 + iota16
                new = []
                for i in range(BPC):
                    m = t == rb[i]
                    cnt = plsc.all_reduce_population_count(m)[0]
                    plsc.store_compressed(lists[i].at[pl.ds(offs[i], 16)], gid, mask=m)
                    new.append(jnp.minimum(offs[i] + cnt, CAP))
                return tuple(new)

            return lax.fori_loop(0, VPC, vec_body, offs)

        z = jnp.int32(0)
        et_start(et_buf, 0, sem)

        def pair_body(p, offs):
            et_start(et_buf2, 2 * p + 1, sem2)
            et_wait(et_buf, sem)
            offs = scan_chunk(et_buf, 2 * p, offs)
            et_start(et_buf, jnp.minimum(2 * p + 2, NCH - 2), sem)
            et_wait(et_buf2, sem2)
            return scan_chunk(et_buf2, 2 * p + 1, offs)

        offs = lax.fori_loop(0, NCH // 2, pair_body, (z, z, z, z))
        et_wait(et_buf, sem)

        for i in range(BPC):
            b = BPC * c + i
            nb = offs[i]
            li = lists[i]
            nch = (nb + GC - 1) // GC

            def fire_sd(k, _):
                idx = li.at[pl.ds(k * GC, GC)]
                pltpu.make_async_copy(src_hbm.at[idx], srcb.at[pl.ds(k * GC, GC)], sem).start()
                pltpu.make_async_copy(dst_hbm.at[idx], dstb.at[pl.ds(k * GC, GC)], sem).start()
                return 0

            def drain_sd(k, _):
                idx = li.at[pl.ds(k * GC, GC)]
                pltpu.make_async_copy(src_hbm.at[idx], srcb.at[pl.ds(k * GC, GC)], sem).wait()
                pltpu.make_async_copy(dst_hbm.at[idx], dstb.at[pl.ds(k * GC, GC)], sem).wait()
                return 0

            lax.fori_loop(0, nch, fire_sd, 0)
            lax.fori_loop(0, nch, drain_sd, 0)

            boff = b * NNODES

            def abs_body(j, _):
                srcb[pl.ds(j * 16, 16)] = srcb[pl.ds(j * 16, 16)] + boff
                return 0

            lax.fori_loop(0, (nb + 15) // 16, abs_body, 0)

            def fire_h(k, _):
                idx = srcb.at[pl.ds(k * GC, GC)]
                pltpu.make_async_copy(h_hbm.at[idx], hb.at[pl.ds(k * GC, GC)], sem).start()
                return 0

            def drain_h(k, _):
                idx = srcb.at[pl.ds(k * GC, GC)]
                pltpu.make_async_copy(h_hbm.at[idx], hb.at[pl.ds(k * GC, GC)], sem).wait()
                return 0

            lax.fori_loop(0, nch, fire_h, 0)
            lax.fori_loop(0, nch, drain_h, 0)

            for seg in range(NSEG):
                lo = seg * SEG

                def zacc_body(j, _):
                    acc[pl.ds(j * 16, 16)] = zeros16
                    return 0

                lax.fori_loop(0, ACCW // 16, zacc_body, 0)

                def scat_body(j, _):
                    d = dstb[pl.ds(j * 16, 16)]
                    v = hb[pl.ds(j * 16, 16)]
                    valid = (j * 16 + iota16 < nb) & (d >= lo) & (d < lo + SEG)
                    loc = jnp.where(valid, d - lo, 0)
                    veff = jnp.where(valid, v, -1.0)

                    def wbody(_):
                        cur = plsc.load_gather(acc, [loc])
                        upd = veff > cur
                        plsc.store_scatter(acc, [loc], veff, mask=upd)
                        cur2 = plsc.load_gather(acc, [loc])
                        return jnp.any(veff > cur2)

                    lax.while_loop(lambda p: p, wbody, jnp.bool_(True))
                    return 0

                lax.fori_loop(0, (nb + 15) // 16, scat_body, 0)

                pltpu.sync_copy(acc, shared.at[pl.ds(pl.multiple_of(s * ACCW, 8), ACCW)])
                plsc.subcore_barrier()

                roff = pl.multiple_of(jnp.where(s < NSUB - 1, s * SL, LAST_OFF), 8)
                pltpu.sync_copy(shared.at[pl.ds(roff, SL)], red)
                for t in range(1, NSUB):
                    pltpu.sync_copy(shared.at[pl.ds(pl.multiple_of(t * ACCW + roff, 8), SL)], tmp)

                    def rmax_body(j, _):
                        red[pl.ds(j * 16, 16)] = jnp.maximum(
                            red[pl.ds(j * 16, 16)], tmp[pl.ds(j * 16, 16)])
                        return 0

                    lax.fori_loop(0, SL // 16, rmax_body, 0)

                out_off = pl.multiple_of(b * NNODES + lo + roff, 8)
                pltpu.sync_copy(red, out_hbm.at[pl.ds(out_off, SL)])
                plsc.subcore_barrier()

    return body(h_flat, src, dst, edge_type, r16)


def kernel(h_prob, edge_index, edge_type, r_index):
    h_flat = h_prob.reshape(-1)
    src = edge_index[0]
    dst = edge_index[1]
    r16 = jnp.concatenate([r_index, jnp.zeros((16 - BATCH,), jnp.int32)])
    out = _sc_traversal(h_flat, src, dst, edge_type, r16)
    return out.reshape(BATCH, NNODES)
